# Initial kernel scaffold; baseline (speedup 1.0000x reference)
import jax
import jax.numpy as jnp
from jax.experimental import pallas as pl


def kernel(X, emb_e, emb_r):
    raise NotImplementedError("write your pallas kernel here")


# speedup vs baseline: 2.3930x; 2.3930x over previous
import functools

import jax
import jax.numpy as jnp
from jax import lax
from jax.experimental import pallas as pl
from jax.experimental.pallas import tpu as pltpu
from jax.experimental.pallas import tpu_sc as plsc

NC, NS = 2, 16
NW = NC * NS
B = 16384
D = 128
CHUNK = 128
ROWS_PER_W = B // NW
CH_PER_W = ROWS_PER_W // CHUNK

_mesh = plsc.VectorSubcoreMesh(core_axis_name="c", subcore_axis_name="s",
                               num_cores=NC, num_subcores=NS)


@functools.partial(
    pl.kernel,
    out_type=jax.ShapeDtypeStruct((B, 3 * D), jnp.float32),
    mesh=_mesh,
    scratch_types=[
        pltpu.VMEM((3, CH_PER_W, CHUNK), jnp.int32),
        pltpu.VMEM((CHUNK, D), jnp.float32),
        pltpu.SemaphoreType.DMA,
    ],
)
def _lookup(idx_hbm, emb_e_hbm, emb_r_hbm, out_hbm, idx_v, rows_v, sem):
    wid = lax.axis_index("s") * NC + lax.axis_index("c")
    base = wid * ROWS_PER_W
    pltpu.sync_copy(idx_hbm.at[wid], idx_v)
    for t in range(3):
        table = emb_r_hbm if t == 1 else emb_e_hbm
        for j in range(CH_PER_W):
            pltpu.async_copy(table.at[idx_v.at[t, j]], rows_v, sem).wait()
            pltpu.sync_copy(
                rows_v,
                out_hbm.at[pl.ds(base + j * CHUNK, CHUNK), pl.ds(t * D, D)])


def kernel(X, emb_e, emb_r):
    idx = X.T.reshape(3, NW, CH_PER_W, CHUNK).transpose(1, 0, 2, 3)
    return _lookup(idx, emb_e, emb_r)


# speedup vs baseline: 2.6744x; 1.1176x over previous
import functools

import jax
import jax.numpy as jnp
from jax import lax
from jax.experimental import pallas as pl
from jax.experimental.pallas import tpu as pltpu
from jax.experimental.pallas import tpu_sc as plsc

NC, NS = 2, 16
NW = NC * NS
B = 16384
D = 128
CHUNK = 128
ROWS_PER_W = B // NW
CH_PER_W = ROWS_PER_W // CHUNK

_mesh = plsc.VectorSubcoreMesh(core_axis_name="c", subcore_axis_name="s",
                               num_cores=NC, num_subcores=NS)


NBUF = 4
NCHUNKS = 3 * CH_PER_W


@functools.partial(
    pl.kernel,
    out_type=jax.ShapeDtypeStruct((B, 3 * D), jnp.float32),
    mesh=_mesh,
    scratch_types=[
        pltpu.VMEM((3, CH_PER_W, CHUNK), jnp.int32),
        pltpu.VMEM((NBUF, CHUNK, D), jnp.float32),
        pltpu.SemaphoreType.DMA((NBUF,)),
        pltpu.SemaphoreType.DMA((NBUF,)),
    ],
)
def _lookup(idx_hbm, emb_e_hbm, emb_r_hbm, out_hbm, idx_v, rows_v, gsem, wsem):
    wid = lax.axis_index("s") * NC + lax.axis_index("c")
    base = wid * ROWS_PER_W
    pltpu.sync_copy(idx_hbm.at[wid], idx_v)

    def gather(i, b):
        t, j = divmod(i, CH_PER_W)
        table = emb_r_hbm if t == 1 else emb_e_hbm
        return pltpu.async_copy(table.at[idx_v.at[t, j]], rows_v.at[b],
                                gsem.at[b])

    def writeback(i, b):
        t, j = divmod(i, CH_PER_W)
        dst = out_hbm.at[pl.ds(base + j * CHUNK, CHUNK), pl.ds(t * D, D)]
        return pltpu.async_copy(rows_v.at[b], dst, wsem.at[b])

    g = [None] * NCHUNKS
    w = [None] * NCHUNKS
    for i in range(NBUF):
        g[i] = gather(i, i)
    for i in range(NCHUNKS):
        b = i % NBUF
        g[i].wait()
        w[i] = writeback(i, b)
        nxt = i + NBUF
        if nxt < NCHUNKS:
            w[i].wait()
            g[nxt] = gather(nxt, b)
    for i in range(NCHUNKS - NBUF, NCHUNKS):
        w[i].wait()


def kernel(X, emb_e, emb_r):
    idx = X.T.reshape(3, NW, CH_PER_W, CHUNK).transpose(1, 0, 2, 3)
    return _lookup(idx, emb_e, emb_r)
